# baseline (speedup 1.0000x reference)
import functools

import jax
import jax.numpy as jnp
from jax import lax
from jax.experimental import pallas as pl
from jax.experimental.pallas import tpu as pltpu
from jax.experimental.pallas import tpu_sc as plsc

EMBED = 128
L = 20
NCLS = 28
NCORES = 2
NSUB = 16
NWORKERS = NCORES * NSUB
GCHUNK = 128
KBUF = 4
LANES = 16


def _sc_gather_pool(table, idx2d, seg2d, n_rows, batch):
    rows_per_w = n_rows // NWORKERS
    b_per_w = batch // NWORKERS
    nch = rows_per_w // GCHUNK
    nph = 2
    bpp = b_per_w // nph
    nch_p = nch // nph
    nit_p = nch_p // KBUF
    mesh = plsc.VectorSubcoreMesh(core_axis_name="c", subcore_axis_name="s")

    @functools.partial(
        pl.kernel,
        out_type=jax.ShapeDtypeStruct((batch, EMBED), jnp.float32),
        mesh=mesh,
        scratch_types=[
            pltpu.VMEM((nch, GCHUNK), jnp.int32),
            pltpu.VMEM((nch, GCHUNK), jnp.int32),
            pltpu.VMEM_SHARED((NSUB * bpp, EMBED), jnp.float32),
        ]
        + [pltpu.VMEM((GCHUNK, EMBED), jnp.float32) for _ in range(KBUF)]
        + [pltpu.SemaphoreType.DMA for _ in range(KBUF)],
    )
    def k(table_hbm, idx_hbm, seg_hbm, out_hbm, idx_v, seg_v, acc_sh,
          *bufs_sem):
        bufs, sems = bufs_sem[:KBUF], bufs_sem[KBUF:]
        c = lax.axis_index("c")
        s = lax.axis_index("s")
        w = c * NSUB + s
        pltpu.sync_copy(idx_hbm.at[pl.ds(w * nch, nch)], idx_v)
        pltpu.sync_copy(seg_hbm.at[s], seg_v)

        zeros = jnp.zeros((LANES,), jnp.float32)

        for ph in range(nph):
            @pl.loop(0, GCHUNK)
            def _(r):
                for u in range(EMBED // LANES):
                    bufs[0][r, pl.ds(u * LANES, LANES)] = zeros

            for t in range(bpp // GCHUNK):
                pltpu.sync_copy(bufs[0],
                                acc_sh.at[pl.ds(s * bpp + t * GCHUNK, GCHUNK)])

            @pl.loop(0, nit_p)
            def _(jj):
                base_c = ph * nch_p + jj * KBUF
                cps = [
                    pltpu.async_copy(table_hbm.at[idx_v.at[base_c + p]],
                                     bufs[p], sems[p])
                    for p in range(KBUF)
                ]
                for p in range(KBUF):
                    cps[p].wait()
                    pltpu.sync_copy(bufs[p], acc_sh.at[seg_v.at[base_c + p]],
                                    add=True)

            plsc.subcore_barrier()
            pltpu.sync_copy(acc_sh.at[pl.ds(s * bpp, bpp)],
                            out_hbm.at[pl.ds(w * b_per_w + ph * bpp, bpp)])

    return k(table, idx2d, seg2d)


def _tc_mlp(pooled_sum, W1, b1, W2, b2, batch):
    BB = 512

    def body(p_ref, w1_ref, b1_ref, w2_ref, b2_ref, o_ref):
        pooled = p_ref[...] * (1.0 / L)
        h = jnp.maximum(jnp.dot(pooled, w1_ref[...],
                                preferred_element_type=jnp.float32) + b1_ref[...], 0.0)
        o_ref[...] = jnp.dot(h, w2_ref[...],
                             preferred_element_type=jnp.float32) + b2_ref[...]

    return pl.pallas_call(
        body,
        grid=(batch // BB,),
        in_specs=[
            pl.BlockSpec((BB, EMBED), lambda i: (i, 0)),
            pl.BlockSpec((EMBED, W1.shape[1]), lambda i: (0, 0)),
            pl.BlockSpec((1, W1.shape[1]), lambda i: (0, 0)),
            pl.BlockSpec((W1.shape[1], NCLS), lambda i: (0, 0)),
            pl.BlockSpec((1, NCLS), lambda i: (0, 0)),
        ],
        out_specs=pl.BlockSpec((BB, NCLS), lambda i: (i, 0)),
        out_shape=jax.ShapeDtypeStruct((batch, NCLS), jnp.float32),
    )(pooled_sum, W1, b1.reshape(1, -1), W2, b2.reshape(1, -1))


def kernel(x, table, W1, b1, W2, b2):
    batch, seq = x.shape
    n_rows = batch * seq
    rows_per_w = n_rows // NWORKERS
    b_per_w = batch // NWORKERS
    idx2d = x.astype(jnp.int32).reshape(n_rows // GCHUNK, GCHUNK)
    bpp = b_per_w // 2
    local_seg = (jnp.arange(rows_per_w, dtype=jnp.int32) // seq) % bpp
    seg2d = (jnp.arange(NSUB, dtype=jnp.int32)[:, None] * bpp
             + local_seg[None, :]).reshape(NSUB, rows_per_w // GCHUNK, GCHUNK)
    pooled_sum = _sc_gather_pool(table, idx2d, seg2d, n_rows, batch)
    return _tc_mlp(pooled_sum, W1, b1, W2, b2, batch)


# speedup vs baseline: 1.1959x; 1.1959x over previous
import functools

import jax
import jax.numpy as jnp
from jax import lax
from jax.experimental import pallas as pl
from jax.experimental.pallas import tpu as pltpu
from jax.experimental.pallas import tpu_sc as plsc

EMBED = 128
L = 20
NCLS = 28
NCORES = 2
NSUB = 16
NWORKERS = NCORES * NSUB
GCHUNK = 128
KBUF = 4
LANES = 16


def _sc_gather_pool(table, idx2d, seg2d, n_rows, batch):
    rows_per_w = n_rows // NWORKERS
    b_per_w = batch // NWORKERS
    nch = rows_per_w // GCHUNK
    nph = 2
    bpp = b_per_w // nph
    nch_p = nch // nph
    nit_p = nch_p // KBUF
    mesh = plsc.VectorSubcoreMesh(core_axis_name="c", subcore_axis_name="s")

    @functools.partial(
        pl.kernel,
        out_type=jax.ShapeDtypeStruct((batch, EMBED), jnp.float32),
        mesh=mesh,
        scratch_types=[
            pltpu.VMEM((nch, GCHUNK), jnp.int32),
            pltpu.VMEM((nch, GCHUNK), jnp.int32),
            pltpu.VMEM_SHARED((NSUB * bpp, EMBED), jnp.float32),
        ]
        + [pltpu.VMEM((GCHUNK, EMBED), jnp.float32) for _ in range(KBUF)]
        + [pltpu.SemaphoreType.DMA for _ in range(KBUF)],
    )
    def k(table_hbm, idx_hbm, seg_hbm, out_hbm, idx_v, seg_v, acc_sh,
          *bufs_sem):
        bufs, sems = bufs_sem[:KBUF], bufs_sem[KBUF:]
        c = lax.axis_index("c")
        s = lax.axis_index("s")
        w = c * NSUB + s
        pltpu.sync_copy(idx_hbm.at[pl.ds(w * nch, nch)], idx_v)
        pltpu.sync_copy(seg_hbm.at[s], seg_v)

        zeros = jnp.zeros((LANES,), jnp.float32)

        for ph in range(nph):
            @pl.loop(0, GCHUNK)
            def _(r):
                for u in range(EMBED // LANES):
                    bufs[0][r, pl.ds(u * LANES, LANES)] = zeros

            for t in range(bpp // GCHUNK):
                pltpu.sync_copy(bufs[0],
                                acc_sh.at[pl.ds(s * bpp + t * GCHUNK, GCHUNK)])

            ph_base = ph * nch_p
            for p in range(KBUF):
                pltpu.async_copy(table_hbm.at[idx_v.at[ph_base + p]],
                                 bufs[p], sems[p])

            @pl.loop(0, nit_p)
            def _(jj):
                base_c = ph_base + jj * KBUF
                for p in range(KBUF):
                    pltpu.make_async_copy(
                        table_hbm.at[idx_v.at[base_c + p]], bufs[p],
                        sems[p]).wait()
                    pltpu.sync_copy(bufs[p], acc_sh.at[seg_v.at[base_c + p]],
                                    add=True)

                    @pl.when(jj < nit_p - 1)
                    def _():
                        pltpu.async_copy(
                            table_hbm.at[idx_v.at[base_c + KBUF + p]],
                            bufs[p], sems[p])

            plsc.subcore_barrier()
            pltpu.sync_copy(acc_sh.at[pl.ds(s * bpp, bpp)],
                            out_hbm.at[pl.ds(w * b_per_w + ph * bpp, bpp)])

    return k(table, idx2d, seg2d)


def _tc_mlp(pooled_sum, W1, b1, W2, b2, batch):
    BB = 512

    def body(p_ref, w1_ref, b1_ref, w2_ref, b2_ref, o_ref):
        pooled = (p_ref[...] * (1.0 / L)).astype(jnp.bfloat16)
        h = jnp.maximum(jnp.dot(pooled, w1_ref[...].astype(jnp.bfloat16),
                                preferred_element_type=jnp.float32) + b1_ref[...], 0.0)
        o_ref[...] = jnp.dot(h.astype(jnp.bfloat16),
                             w2_ref[...].astype(jnp.bfloat16),
                             preferred_element_type=jnp.float32) + b2_ref[...]

    return pl.pallas_call(
        body,
        grid=(batch // BB,),
        in_specs=[
            pl.BlockSpec((BB, EMBED), lambda i: (i, 0)),
            pl.BlockSpec((EMBED, W1.shape[1]), lambda i: (0, 0)),
            pl.BlockSpec((1, W1.shape[1]), lambda i: (0, 0)),
            pl.BlockSpec((W1.shape[1], NCLS), lambda i: (0, 0)),
            pl.BlockSpec((1, NCLS), lambda i: (0, 0)),
        ],
        out_specs=pl.BlockSpec((BB, NCLS), lambda i: (i, 0)),
        out_shape=jax.ShapeDtypeStruct((batch, NCLS), jnp.float32),
    )(pooled_sum, W1, b1.reshape(1, -1), W2, b2.reshape(1, -1))


def kernel(x, table, W1, b1, W2, b2):
    batch, seq = x.shape
    n_rows = batch * seq
    rows_per_w = n_rows // NWORKERS
    b_per_w = batch // NWORKERS
    idx2d = x.astype(jnp.int32).reshape(n_rows // GCHUNK, GCHUNK)
    bpp = b_per_w // 2
    local_seg = (jnp.arange(rows_per_w, dtype=jnp.int32) // seq) % bpp
    seg2d = (jnp.arange(NSUB, dtype=jnp.int32)[:, None] * bpp
             + local_seg[None, :]).reshape(NSUB, rows_per_w // GCHUNK, GCHUNK)
    pooled_sum = _sc_gather_pool(table, idx2d, seg2d, n_rows, batch)
    return _tc_mlp(pooled_sum, W1, b1, W2, b2, batch)
